# Initial kernel scaffold; baseline (speedup 1.0000x reference)
import jax
import jax.numpy as jnp
from jax.experimental import pallas as pl


def kernel(img, z):
    raise NotImplementedError("write your pallas kernel here")


# speedup vs baseline: 143.4636x; 143.4636x over previous
import functools

import jax
import jax.numpy as jnp
from jax import lax
from jax.experimental import pallas as pl
from jax.experimental.pallas import tpu as pltpu
from jax.experimental.pallas import tpu_sc as plsc

NPTS = 262144
NC, NS = 2, 16
NW = NC * NS
PPW = NPTS // NW
CH = 32
NCHUNK = PPW // CH
SLICE = NPTS // NS
IMG_N = 512

_mesh = plsc.VectorSubcoreMesh(core_axis_name="c", subcore_axis_name="s")


@functools.partial(
    pl.kernel,
    mesh=_mesh,
    out_type=jax.ShapeDtypeStruct((NC, 2, NPTS), jnp.float32),
    scratch_types=[
        pltpu.VMEM((PPW,), jnp.float32),
        pltpu.VMEM((PPW,), jnp.float32),
        pltpu.VMEM((4 * CH,), jnp.int32),
        pltpu.VMEM((4 * CH,), jnp.float32),
        pltpu.VMEM((4 * CH,), jnp.float32),
        pltpu.VMEM((SLICE,), jnp.float32),
        pltpu.VMEM_SHARED((NPTS,), jnp.float32),
        pltpu.VMEM_SHARED((NPTS,), jnp.float32),
    ],
)
def _sc_fields(z0_hbm, z1_hbm, zero_hbm, out_hbm,
               z0_v, z1_v, idx_v, vx_v, vy_v, buf_v, a0_sh, a1_sh):
    c = lax.axis_index("c")
    s = lax.axis_index("s")
    wid = s * NC + c

    pltpu.sync_copy(zero_hbm, buf_v)
    pltpu.sync_copy(buf_v, a0_sh.at[pl.ds(s * SLICE, SLICE)])
    pltpu.sync_copy(buf_v, a1_sh.at[pl.ds(s * SLICE, SLICE)])
    plsc.subcore_barrier()

    base = wid * PPW
    pltpu.sync_copy(z0_hbm.at[pl.ds(base, PPW)], z0_v)
    pltpu.sync_copy(z1_hbm.at[pl.ds(base, PPW)], z1_v)

    def chunk(k, _):
        for j in range(CH // 16):
            off = k * CH + j * 16
            yf = z0_v[pl.ds(off, 16)] * float(IMG_N - 1)
            xf = z1_v[pl.ds(off, 16)] * float(IMG_N - 1)
            y = yf.astype(jnp.int32)
            x = xf.astype(jnp.int32)
            fx = x.astype(jnp.float32) - xf
            fy = y.astype(jnp.float32) - yf
            p = y * IMG_N + x
            one = jnp.float32(1.0)
            for g, (dp, wx, wy) in enumerate((
                    (0, -one - fx, -one - fy),
                    (1, fx, one + fy),
                    (IMG_N, one + fx, fy),
                    (IMG_N + 1, -fx, -fy),
            )):
                r = g * CH + j * 16
                idx_v[pl.ds(r, 16)] = p + dp
                vx_v[pl.ds(r, 16)] = wx
                vy_v[pl.ds(r, 16)] = wy
        pltpu.sync_copy(vx_v, a0_sh.at[idx_v], add=True)
        pltpu.sync_copy(vy_v, a1_sh.at[idx_v], add=True)
        return 0

    lax.fori_loop(0, NCHUNK, chunk, 0)

    plsc.subcore_barrier()
    pltpu.sync_copy(a0_sh.at[pl.ds(s * SLICE, SLICE)], buf_v)
    pltpu.sync_copy(buf_v, out_hbm.at[c, 0, pl.ds(s * SLICE, SLICE)])
    pltpu.sync_copy(a1_sh.at[pl.ds(s * SLICE, SLICE)], buf_v)
    pltpu.sync_copy(buf_v, out_hbm.at[c, 1, pl.ds(s * SLICE, SLICE)])


_KB = 4096
_GRID = NPTS // _KB


def _tc_body(img_ref, a0_ref, a1_ref, out_ref):
    k = pl.program_id(0)
    blk = img_ref[...]
    p0 = jnp.sum(blk * a0_ref[...], axis=1, keepdims=True)
    p1 = jnp.sum(blk * a1_ref[...], axis=1, keepdims=True)
    part = jnp.concatenate([p0, p1], axis=1)

    @pl.when(k == 0)
    def _():
        out_ref[...] = part

    @pl.when(k > 0)
    def _():
        out_ref[...] += part


_tc_contract = pl.pallas_call(
    _tc_body,
    grid=(_GRID,),
    in_specs=[
        pl.BlockSpec((24, _KB), lambda k: (0, k)),
        pl.BlockSpec((1, _KB), lambda k: (0, k)),
        pl.BlockSpec((1, _KB), lambda k: (0, k)),
    ],
    out_specs=pl.BlockSpec((24, 2), lambda k: (0, 0)),
    out_shape=jax.ShapeDtypeStruct((24, 2), jnp.float32),
)


def kernel(img, z):
    z0 = z[:, 0]
    z1 = z[:, 1]
    zeros_tile = jnp.zeros((SLICE,), jnp.float32)
    a2 = _sc_fields(z0, z1, zeros_tile)
    a = a2[0] + a2[1]
    a0 = a[0].reshape(1, NPTS)
    a1 = a[1].reshape(1, NPTS)
    img2 = img.reshape(24, NPTS)
    out24 = _tc_contract(img2, a0, a1)
    return out24.reshape(8, 3, 2).sum(axis=1)


# speedup vs baseline: 191.2578x; 1.3331x over previous
import functools

import jax
import jax.numpy as jnp
from jax import lax
from jax.experimental import pallas as pl
from jax.experimental.pallas import tpu as pltpu
from jax.experimental.pallas import tpu_sc as plsc

NPTS = 262144
NC, NS = 2, 16
NW = NC * NS
PPW = NPTS // NW
CH = 32
NCHUNK = PPW // CH
DEPTH = 8
SLICE = NPTS // NS
IMG_N = 512

_mesh = plsc.VectorSubcoreMesh(core_axis_name="c", subcore_axis_name="s")


@functools.partial(
    pl.kernel,
    mesh=_mesh,
    out_type=jax.ShapeDtypeStruct((NC, 2, NPTS), jnp.float32),
    scratch_types=[
        pltpu.VMEM((PPW,), jnp.float32),
        pltpu.VMEM((PPW,), jnp.float32),
        pltpu.VMEM((DEPTH, 4 * CH), jnp.int32),
        pltpu.VMEM((DEPTH, 4 * CH), jnp.float32),
        pltpu.VMEM((DEPTH, 4 * CH), jnp.float32),
        pltpu.VMEM((SLICE,), jnp.float32),
        pltpu.VMEM_SHARED((NPTS,), jnp.float32),
        pltpu.VMEM_SHARED((NPTS,), jnp.float32),
        pltpu.SemaphoreType.DMA,
    ],
)
def _sc_fields(z0_hbm, z1_hbm, zero_hbm, out_hbm,
               z0_v, z1_v, idx_v, vx_v, vy_v, buf_v, a0_sh, a1_sh, sem):
    c = lax.axis_index("c")
    s = lax.axis_index("s")
    wid = s * NC + c

    pltpu.sync_copy(zero_hbm, buf_v)
    pltpu.sync_copy(buf_v, a0_sh.at[pl.ds(s * SLICE, SLICE)])
    pltpu.sync_copy(buf_v, a1_sh.at[pl.ds(s * SLICE, SLICE)])
    plsc.subcore_barrier()

    pltpu.sync_copy(z0_hbm.at[pl.ds(wid * PPW, PPW)], z0_v)
    pltpu.sync_copy(z1_hbm.at[pl.ds(wid * PPW, PPW)], z1_v)

    def build_chunk(k, d):
        for j in range(CH // 16):
            off = k * CH + j * 16
            yf = z0_v[pl.ds(off, 16)] * float(IMG_N - 1)
            xf = z1_v[pl.ds(off, 16)] * float(IMG_N - 1)
            y = yf.astype(jnp.int32)
            x = xf.astype(jnp.int32)
            fx = x.astype(jnp.float32) - xf
            fy = y.astype(jnp.float32) - yf
            p = y * IMG_N + x
            one = jnp.float32(1.0)
            for g, (dp, wx, wy) in enumerate((
                    (0, -one - fx, -one - fy),
                    (1, fx, one + fy),
                    (IMG_N, one + fx, fy),
                    (IMG_N + 1, -fx, -fy),
            )):
                r = g * CH + j * 16
                idx_v[d, pl.ds(r, 16)] = p + dp
                vx_v[d, pl.ds(r, 16)] = wx
                vy_v[d, pl.ds(r, 16)] = wy

    def fire(d):
        h0 = pltpu.async_copy(vx_v.at[d], a0_sh.at[idx_v.at[d]], sem, add=True)
        h1 = pltpu.async_copy(vy_v.at[d], a1_sh.at[idx_v.at[d]], sem, add=True)
        return h0, h1

    def round_(kk, _):
        handles = []
        for d in range(DEPTH):
            build_chunk(kk * DEPTH + d, d)
            handles.append(fire(d))
        for h0, h1 in handles:
            h0.wait()
            h1.wait()
        return 0

    lax.fori_loop(0, NCHUNK // DEPTH, round_, 0)

    plsc.subcore_barrier()
    pltpu.sync_copy(a0_sh.at[pl.ds(s * SLICE, SLICE)], buf_v)
    pltpu.sync_copy(buf_v, out_hbm.at[c, 0, pl.ds(s * SLICE, SLICE)])
    pltpu.sync_copy(a1_sh.at[pl.ds(s * SLICE, SLICE)], buf_v)
    pltpu.sync_copy(buf_v, out_hbm.at[c, 1, pl.ds(s * SLICE, SLICE)])


_KB = 4096
_GRID = NPTS // _KB


def _tc_body(img_ref, a_ref, out_ref):
    k = pl.program_id(0)
    blk = img_ref[...]
    a = a_ref[...]
    a0 = a[0, 0, :] + a[1, 0, :]
    a1 = a[0, 1, :] + a[1, 1, :]
    p0 = jnp.sum(blk * a0[None, :], axis=1, keepdims=True)
    p1 = jnp.sum(blk * a1[None, :], axis=1, keepdims=True)
    part = jnp.concatenate([p0, p1], axis=1)

    @pl.when(k == 0)
    def _():
        out_ref[...] = part

    @pl.when(k > 0)
    def _():
        out_ref[...] += part


_tc_contract = pl.pallas_call(
    _tc_body,
    grid=(_GRID,),
    in_specs=[
        pl.BlockSpec((24, _KB), lambda k: (0, k)),
        pl.BlockSpec((2, 2, _KB), lambda k: (0, 0, k)),
    ],
    out_specs=pl.BlockSpec((24, 2), lambda k: (0, 0)),
    out_shape=jax.ShapeDtypeStruct((24, 2), jnp.float32),
)


def kernel(img, z):
    zeros_tile = jnp.zeros((SLICE,), jnp.float32)
    a2 = _sc_fields(z[:, 0], z[:, 1], zeros_tile)
    img2 = img.reshape(24, NPTS)
    out24 = _tc_contract(img2, a2)
    return out24.reshape(8, 3, 2).sum(axis=1)


# speedup vs baseline: 191.8115x; 1.0029x over previous
import functools

import jax
import jax.numpy as jnp
from jax import lax
from jax.experimental import pallas as pl
from jax.experimental.pallas import tpu as pltpu
from jax.experimental.pallas import tpu_sc as plsc

NPTS = 262144
NC, NS = 2, 16
NW = NC * NS
PPW = NPTS // NW
CH = 32
NCHUNK = PPW // CH
DEPTH = 8
SLICE = NPTS // NS
IMG_N = 512

_mesh = plsc.VectorSubcoreMesh(core_axis_name="c", subcore_axis_name="s")


@functools.partial(
    pl.kernel,
    mesh=_mesh,
    out_type=jax.ShapeDtypeStruct((NC, 2, NPTS), jnp.float32),
    scratch_types=[
        pltpu.VMEM((PPW,), jnp.float32),
        pltpu.VMEM((PPW,), jnp.float32),
        pltpu.VMEM((DEPTH, 4 * CH), jnp.int32),
        pltpu.VMEM((DEPTH, 4 * CH), jnp.float32),
        pltpu.VMEM((DEPTH, 4 * CH), jnp.float32),
        pltpu.VMEM((SLICE,), jnp.float32),
        pltpu.VMEM_SHARED((NPTS,), jnp.float32),
        pltpu.VMEM_SHARED((NPTS,), jnp.float32),
        pltpu.SemaphoreType.DMA,
    ],
)
def _sc_fields(zt_hbm, zero_hbm, out_hbm,
               z0_v, z1_v, idx_v, vx_v, vy_v, buf_v, a0_sh, a1_sh, sem):
    c = lax.axis_index("c")
    s = lax.axis_index("s")
    wid = s * NC + c

    pltpu.sync_copy(zero_hbm, buf_v)
    pltpu.sync_copy(buf_v, a0_sh.at[pl.ds(s * SLICE, SLICE)])
    pltpu.sync_copy(buf_v, a1_sh.at[pl.ds(s * SLICE, SLICE)])
    plsc.subcore_barrier()

    pltpu.sync_copy(zt_hbm.at[0, pl.ds(wid * PPW, PPW)], z0_v)
    pltpu.sync_copy(zt_hbm.at[1, pl.ds(wid * PPW, PPW)], z1_v)

    def build_chunk(k, d):
        for j in range(CH // 16):
            off = k * CH + j * 16
            yf = z0_v[pl.ds(off, 16)] * float(IMG_N - 1)
            xf = z1_v[pl.ds(off, 16)] * float(IMG_N - 1)
            y = yf.astype(jnp.int32)
            x = xf.astype(jnp.int32)
            fx = x.astype(jnp.float32) - xf
            fy = y.astype(jnp.float32) - yf
            p = y * IMG_N + x
            one = jnp.float32(1.0)
            for g, (dp, wx, wy) in enumerate((
                    (0, -one - fx, -one - fy),
                    (1, fx, one + fy),
                    (IMG_N, one + fx, fy),
                    (IMG_N + 1, -fx, -fy),
            )):
                r = g * CH + j * 16
                idx_v[d, pl.ds(r, 16)] = p + dp
                vx_v[d, pl.ds(r, 16)] = wx
                vy_v[d, pl.ds(r, 16)] = wy

    def fire(d):
        h0 = pltpu.async_copy(vx_v.at[d], a0_sh.at[idx_v.at[d]], sem, add=True)
        h1 = pltpu.async_copy(vy_v.at[d], a1_sh.at[idx_v.at[d]], sem, add=True)
        return h0, h1

    def round_(kk, _):
        handles = []
        for d in range(DEPTH):
            build_chunk(kk * DEPTH + d, d)
            handles.append(fire(d))
        for h0, h1 in handles:
            h0.wait()
            h1.wait()
        return 0

    lax.fori_loop(0, NCHUNK // DEPTH, round_, 0)

    plsc.subcore_barrier()
    pltpu.sync_copy(a0_sh.at[pl.ds(s * SLICE, SLICE)], buf_v)
    pltpu.sync_copy(buf_v, out_hbm.at[c, 0, pl.ds(s * SLICE, SLICE)])
    pltpu.sync_copy(a1_sh.at[pl.ds(s * SLICE, SLICE)], buf_v)
    pltpu.sync_copy(buf_v, out_hbm.at[c, 1, pl.ds(s * SLICE, SLICE)])


_KB = 4096
_GRID = NPTS // _KB


def _tc_body(img_ref, a_ref, out_ref):
    k = pl.program_id(0)
    blk = img_ref[...]
    a = a_ref[...]
    a0 = a[0, 0, :] + a[1, 0, :]
    a1 = a[0, 1, :] + a[1, 1, :]
    p0 = jnp.sum(blk * a0[None, :], axis=1, keepdims=True)
    p1 = jnp.sum(blk * a1[None, :], axis=1, keepdims=True)
    part = jnp.concatenate([p0, p1], axis=1)

    @pl.when(k == 0)
    def _():
        out_ref[...] = part

    @pl.when(k > 0)
    def _():
        out_ref[...] += part


_tc_contract = pl.pallas_call(
    _tc_body,
    grid=(_GRID,),
    in_specs=[
        pl.BlockSpec((24, _KB), lambda k: (0, k)),
        pl.BlockSpec((2, 2, _KB), lambda k: (0, 0, k)),
    ],
    out_specs=pl.BlockSpec((24, 2), lambda k: (0, 0)),
    out_shape=jax.ShapeDtypeStruct((24, 2), jnp.float32),
)


def kernel(img, z):
    zeros_tile = jnp.zeros((SLICE,), jnp.float32)
    a2 = _sc_fields(z.T, zeros_tile)
    img2 = img.reshape(24, NPTS)
    out24 = _tc_contract(img2, a2)
    return out24.reshape(8, 3, 2).sum(axis=1)
